# baseline (speedup 1.0000x reference)
import jax
import jax.numpy as jnp
from jax import lax
from jax.experimental import pallas as pl
from jax.experimental.pallas import tpu as pltpu
from jax.experimental.pallas import tpu_sc as plsc

N = 10000
NP = 10240
E = 320000
D = 128
R = 2
HALF = 64
NC = 2
NS = 16
L = 16
CH = 128
NCH = 160
ET = NCH * CH
EPAD = NS * ET
NCH32 = EPAD // (2 * NS) // CH
NSEG = 4
CHD = 128
NCHD = EPAD // (2 * NS) // CHD
SEGC = NCH // NSEG
SEGE = SEGC * CH


def _mesh():
    return plsc.VectorSubcoreMesh(
        core_axis_name="c", subcore_axis_name="s", num_cores=NC, num_subcores=NS
    )


def _deg_body(x3_hbm, y3_hbm, z_hbm, out_hbm,
              hist, Xb, Cb, ones, sa0, sb0, sa1, sb1):
    cid = lax.axis_index("c")
    sid = lax.axis_index("s")
    wid = cid * NS + sid
    for j in range(CH // L):
        ones[pl.ds(j * L, L)] = jnp.ones((L,), jnp.float32)

    zsl = pl.ds(sid * (4 * NP // NS), 4 * NP // NS)
    pltpu.sync_copy(z_hbm.at[zsl], hist.at[zsl])

    et32 = NCH32 * CH
    pltpu.sync_copy(x3_hbm.at[pl.ds(wid * et32, et32)], Xb)
    pltpu.sync_copy(y3_hbm.at[pl.ds(wid * et32, et32)], Cb)

    def cc(kk, carry):
        osl = pl.ds(kk * L, L)
        Cb[osl] = Cb[osl] + 2 * NP
        return carry

    lax.fori_loop(0, et32 // L, cc, 0)
    plsc.subcore_barrier()

    sa = (sa0, sa1)
    sb = (sb0, sb1)

    def pair(i2, carry):
        for b in range(2):
            c = i2 * 2 + b

            @pl.when(i2 >= 1)
            def _():
                pltpu.make_async_copy(
                    ones, hist.at[Xb.at[pl.ds((c - 2) * CH, CH)]], sa[b]).wait()
                pltpu.make_async_copy(
                    ones, hist.at[Cb.at[pl.ds((c - 2) * CH, CH)]], sb[b]).wait()

            pltpu.async_copy(ones, hist.at[Xb.at[pl.ds(c * CH, CH)]], sa[b],
                             add=True)
            pltpu.async_copy(ones, hist.at[Cb.at[pl.ds(c * CH, CH)]], sb[b],
                             add=True)
        return carry

    lax.fori_loop(0, NCH32 // 2, pair, 0)
    for b in range(2):
        c = NCH32 - 2 + b
        pltpu.make_async_copy(ones, hist.at[Xb.at[pl.ds(c * CH, CH)]],
                              sa[b]).wait()
        pltpu.make_async_copy(ones, hist.at[Cb.at[pl.ds(c * CH, CH)]],
                              sb[b]).wait()
    plsc.subcore_barrier()
    pltpu.sync_copy(hist.at[zsl],
                    out_hbm.at[pl.ds(cid * 4 * NP + sid * (4 * NP // NS),
                                     4 * NP // NS)])


def _deg_call(x, y):
    k = pl.kernel(
        _deg_body,
        out_type=jax.ShapeDtypeStruct((NC * 4 * NP,), jnp.float32),
        mesh=_mesh(),
        compiler_params=pltpu.CompilerParams(use_tc_tiling_on_sc=False),
        scratch_types=[
            pltpu.VMEM_SHARED((4 * NP,), jnp.float32),
            pltpu.VMEM((NCH32 * CH,), jnp.int32),
            pltpu.VMEM((NCH32 * CH,), jnp.int32),
            pltpu.VMEM((CH,), jnp.float32),
            pltpu.SemaphoreType.DMA,
            pltpu.SemaphoreType.DMA,
            pltpu.SemaphoreType.DMA,
            pltpu.SemaphoreType.DMA,
        ],
    )
    z = jnp.zeros((4 * NP,), jnp.float32)
    return k(x, y, z).reshape(NC, 4 * NP)


_TB = 256


def _table_body(feat_ref, w_ref, deg_ref, tab_ref, inv_ref):
    f = feat_ref[...]
    for r in range(R):
        d = deg_ref[r] + deg_ref[R + r]
        inv = lax.rsqrt(jnp.maximum(d, 1.0))
        inv_ref[r] = inv
        res = jnp.dot(f, w_ref[r], preferred_element_type=jnp.float32)
        res = res * inv[:, None]
        tab_ref[0, r] = res[:, :HALF]
        tab_ref[1, r] = res[:, HALF:]


def _table_call(feat_p, w, deg4):
    return pl.pallas_call(
        _table_body,
        grid=(NP // _TB,),
        in_specs=[
            pl.BlockSpec((_TB, D), lambda nb: (nb, 0)),
            pl.BlockSpec((R, D, D), lambda nb: (0, 0, 0)),
            pl.BlockSpec((2 * R, _TB), lambda nb: (0, nb)),
        ],
        out_specs=[
            pl.BlockSpec((2, R, _TB, HALF), lambda nb: (0, 0, nb, 0)),
            pl.BlockSpec((R, _TB), lambda nb: (0, nb)),
        ],
        out_shape=[
            jax.ShapeDtypeStruct((2, R, NP, HALF), jnp.float32),
            jax.ShapeDtypeStruct((R, NP), jnp.float32),
        ],
    )(feat_p, w, deg4)


def _agg_body(tc_hbm, tg_hbm, x_hbm, y_hbm, z_hbm, aggc_hbm, aggg_hbm,
              A, Zb, Sb2, rows0, rows1, rows2, rows3,
              gs0, gs1, gs2, gs3, ss0, ss1, ss2, ss3):
    cid = lax.axis_index("c")
    sid = lax.axis_index("s")
    tpw = 2 * NP // NS

    def zero_a():
        zsl = pl.ds(sid * tpw, tpw)
        pltpu.sync_copy(z_hbm.at[zsl], A.at[zsl])

    rowsb = (rows0, rows1, rows2, rows3)
    gsem = (gs0, gs1, gs2, gs3)
    ssem = (ss0, ss1, ss2, ss3)

    def run_phase(tab_hbm, g_hbm, s_hbm, out_hbm):
        def seg(s, carry):
            soff = sid * ET + s * SEGE
            pltpu.sync_copy(g_hbm.at[pl.ds(soff, SEGE)], Zb)
            pltpu.sync_copy(s_hbm.at[pl.ds(soff, SEGE)], Sb2)

            def cz(kk, c2):
                osl = pl.ds(kk * L, L)
                Zb[osl] = Zb[osl] + cid * (2 * NP)
                return c2

            lax.fori_loop(0, SEGE // L, cz, 0)

            for b in range(4):
                pltpu.async_copy(tab_hbm.at[Zb.at[pl.ds(b * CH, CH)]],
                                 rowsb[b], gsem[b])

            def quad(i4, c2):
                c0 = i4 * 4
                for b in range(4):
                    c = c0 + b
                    pltpu.make_async_copy(tab_hbm.at[Zb.at[pl.ds(c * CH, CH)]],
                                          rowsb[b], gsem[b]).wait()
                    pltpu.async_copy(rowsb[b], A.at[Sb2.at[pl.ds(c * CH, CH)]],
                                     ssem[b], add=True)

                @pl.when(i4 + 1 < SEGC // 4)
                def _():
                    for b in range(4):
                        c = c0 + b
                        pltpu.make_async_copy(
                            rowsb[b], A.at[Sb2.at[pl.ds(c * CH, CH)]],
                            ssem[b]).wait()
                        pltpu.async_copy(
                            tab_hbm.at[Zb.at[pl.ds((c + 4) * CH, CH)]],
                            rowsb[b], gsem[b])

                return c2

            lax.fori_loop(0, SEGC // 4, quad, 0)
            for b in range(4):
                c = SEGC - 4 + b
                pltpu.make_async_copy(rowsb[b],
                                      A.at[Sb2.at[pl.ds(c * CH, CH)]],
                                      ssem[b]).wait()
            return carry

        lax.fori_loop(0, NSEG, seg, 0)
        plsc.subcore_barrier()
        roff = sid * tpw
        pltpu.sync_copy(A.at[pl.ds(roff, tpw)],
                        out_hbm.at[pl.ds(cid * 2 * NP + roff, tpw)])
        plsc.subcore_barrier()

    zero_a()
    plsc.subcore_barrier()
    run_phase(tc_hbm, x_hbm, y_hbm, aggc_hbm)
    zero_a()
    plsc.subcore_barrier()
    run_phase(tg_hbm, y_hbm, x_hbm, aggg_hbm)


def _agg_call(tabc, tabg, x, y):
    out = jax.ShapeDtypeStruct((NC * 2 * NP, HALF), jnp.float32)
    k = pl.kernel(
        _agg_body,
        out_type=(out, out),
        mesh=_mesh(),
        compiler_params=pltpu.CompilerParams(use_tc_tiling_on_sc=False),
        scratch_types=[
            pltpu.VMEM_SHARED((2 * NP, HALF), jnp.float32),
            pltpu.VMEM((SEGE,), jnp.int32),
            pltpu.VMEM((SEGE,), jnp.int32),
            pltpu.VMEM((CH, HALF), jnp.float32),
            pltpu.VMEM((CH, HALF), jnp.float32),
            pltpu.VMEM((CH, HALF), jnp.float32),
            pltpu.VMEM((CH, HALF), jnp.float32),
            pltpu.SemaphoreType.DMA,
            pltpu.SemaphoreType.DMA,
            pltpu.SemaphoreType.DMA,
            pltpu.SemaphoreType.DMA,
            pltpu.SemaphoreType.DMA,
            pltpu.SemaphoreType.DMA,
            pltpu.SemaphoreType.DMA,
            pltpu.SemaphoreType.DMA,
        ],
    )
    z = jnp.zeros((2 * NP, HALF), jnp.float32)
    aggc, aggg = k(tabc, tabg, x, y, z)
    return (aggc.reshape(NC, R, NP, HALF), aggg.reshape(NC, R, NP, HALF))


def _proj_body(agg_ref, inv_ref, w_ref, b_ref, out_ref):
    parts = []
    for r in range(R):
        iv = inv_ref[r][:, None]
        for h in range(2):
            parts.append(jnp.maximum(agg_ref[h, r], 0.0) * iv)
    a = jnp.concatenate(parts, axis=1)
    acc = jnp.dot(a, w_ref[...], preferred_element_type=jnp.float32)
    out_ref[...] = jnp.maximum(acc + b_ref[...], 0.0)


def _proj_call(agg4, inv, wfc, bfc):
    return pl.pallas_call(
        _proj_body,
        grid=(NP // _TB,),
        in_specs=[
            pl.BlockSpec((2, R, _TB, HALF), lambda nb: (0, 0, nb, 0)),
            pl.BlockSpec((R, _TB), lambda nb: (0, nb)),
            pl.BlockSpec((R * D, D), lambda nb: (0, 0)),
            pl.BlockSpec((1, D), lambda nb: (0, 0)),
        ],
        out_specs=pl.BlockSpec((_TB, D), lambda nb: (nb, 0)),
        out_shape=jax.ShapeDtypeStruct((NP, D), jnp.float32),
    )(agg4, inv, wfc, bfc.reshape(1, D))


def _dec_body(g_hbm, c_hbm, s2_hbm, d2_hbm, out_hbm,
              Sb, Db, g0, g1, c0b, c1b, P, p0, p1, sg0, sg1, sc0, sc1, sw0, sw1):
    cid = lax.axis_index("c")
    sid = lax.axis_index("s")
    wid = cid * NS + sid
    base = wid * (NCHD * CHD)
    lanes = lax.broadcasted_iota(jnp.int32, (L,), 0)

    pltpu.sync_copy(s2_hbm.at[pl.ds(base, NCHD * CHD)], Sb)
    pltpu.sync_copy(d2_hbm.at[pl.ds(base, NCHD * CHD)], Db)

    gbuf = (g0, g1)
    cbuf = (c0b, c1b)
    pbuf = (p0, p1)
    sg = (sg0, sg1)
    sc = (sc0, sc1)
    sw = (sw0, sw1)
    ipc = CHD

    pltpu.async_copy(g_hbm.at[Sb.at[pl.ds(0, ipc)]], gbuf[0], sg[0])
    pltpu.async_copy(c_hbm.at[Db.at[pl.ds(0, ipc)]], cbuf[0], sc[0])
    pltpu.async_copy(g_hbm.at[Sb.at[pl.ds(ipc, ipc)]], gbuf[1], sg[1])
    pltpu.async_copy(c_hbm.at[Db.at[pl.ds(ipc, ipc)]], cbuf[1], sc[1])

    def pair(i2, carry):
        for b in range(2):
            c = i2 * 2 + b
            pltpu.make_async_copy(g_hbm.at[Sb.at[pl.ds(c * ipc, ipc)]],
                                  gbuf[b], sg[b]).wait()
            pltpu.make_async_copy(c_hbm.at[Db.at[pl.ds(c * ipc, ipc)]],
                                  cbuf[b], sc[b]).wait()

            def edot(e, carry2):
                acc = jnp.zeros((L,), jnp.float32)
                for j in range(D // L):
                    acc = acc + (gbuf[b][e, pl.ds(j * L, L)]
                                 * cbuf[b][e, pl.ds(j * L, L)])
                P[pl.ds(e * L, L)] = acc
                return carry2

            lax.fori_loop(0, CHD, edot, 0)

            @pl.when(i2 >= 1)
            def _():
                pltpu.make_async_copy(
                    pbuf[b], out_hbm.at[pl.ds(base + (c - 2) * CHD, CHD)],
                    sw[b]).wait()

            def red(g, carry2):
                acc = jnp.zeros((L,), jnp.float32)
                rowbase = (g * L + lanes) * L
                for j in range(L):
                    acc = acc + plsc.load_gather(P, [rowbase + j])
                pbuf[b][pl.ds(g * L, L)] = acc
                return carry2

            lax.fori_loop(0, CHD // L, red, 0)
            pltpu.async_copy(pbuf[b], out_hbm.at[pl.ds(base + c * CHD, CHD)],
                             sw[b])

            @pl.when(i2 + 1 < NCHD // 2)
            def _():
                pltpu.async_copy(g_hbm.at[Sb.at[pl.ds((c + 2) * ipc, ipc)]],
                                 gbuf[b], sg[b])
                pltpu.async_copy(c_hbm.at[Db.at[pl.ds((c + 2) * ipc, ipc)]],
                                 cbuf[b], sc[b])

        return carry

    lax.fori_loop(0, NCHD // 2, pair, 0)
    for b in range(2):
        c = NCHD - 2 + b
        pltpu.make_async_copy(pbuf[b], out_hbm.at[pl.ds(base + c * CHD, CHD)],
                              sw[b]).wait()


def _dec_call(gout, cout, ds2, dd2):
    k = pl.kernel(
        _dec_body,
        out_type=jax.ShapeDtypeStruct((EPAD,), jnp.float32),
        mesh=_mesh(),
        compiler_params=pltpu.CompilerParams(
            use_tc_tiling_on_sc=False, needs_layout_passes=False),
        scratch_types=[
            pltpu.VMEM((NCHD * CHD,), jnp.int32),
            pltpu.VMEM((NCHD * CHD,), jnp.int32),
            pltpu.VMEM((CHD, D), jnp.float32),
            pltpu.VMEM((CHD, D), jnp.float32),
            pltpu.VMEM((CHD, D), jnp.float32),
            pltpu.VMEM((CHD, D), jnp.float32),
            pltpu.VMEM((CHD * L,), jnp.float32),
            pltpu.VMEM((CHD,), jnp.float32),
            pltpu.VMEM((CHD,), jnp.float32),
            pltpu.SemaphoreType.DMA,
            pltpu.SemaphoreType.DMA,
            pltpu.SemaphoreType.DMA,
            pltpu.SemaphoreType.DMA,
            pltpu.SemaphoreType.DMA,
            pltpu.SemaphoreType.DMA,
        ],
    )
    return k(gout, cout, ds2, dd2)


def kernel(ufeat, ifeat, enc_edge_index, enc_edge_type, dec_edge_index,
           W_u, W_i, Wfc_u, bfc_u, Wfc_i, bfc_i):
    src = enc_edge_index[0].astype(jnp.int32)
    dst = enc_edge_index[1].astype(jnp.int32)
    typ = enc_edge_type.astype(jnp.int32)

    pad = jnp.full((EPAD - E,), N, jnp.int32)
    x = jnp.concatenate([typ * NP + src, pad])
    y = jnp.concatenate([typ * NP + dst, pad])

    degp = _deg_call(x, y)
    gdeg = degp[:, :2 * NP].reshape(2 * R, NP)
    cdeg = degp[:, 2 * NP:].reshape(2 * R, NP)

    ufp = jnp.pad(ufeat, ((0, NP - N), (0, 0)))
    ifp = jnp.pad(ifeat, ((0, NP - N), (0, 0)))

    tabc4, invg = _table_call(ufp, W_u, gdeg)
    tabg4, invc = _table_call(ifp, W_i, cdeg)

    aggc, aggg = _agg_call(tabc4.reshape(2 * R * NP, HALF),
                           tabg4.reshape(2 * R * NP, HALF), x, y)

    gene_out_p = _proj_call(aggg, invg, Wfc_u, bfc_u)
    cell_out_p = _proj_call(aggc, invc, Wfc_i, bfc_i)

    dsrcp = jnp.concatenate([dec_edge_index[0].astype(jnp.int32), pad])
    ddstp = jnp.concatenate([dec_edge_index[1].astype(jnp.int32), pad])
    pred = _dec_call(gene_out_p, cell_out_p, dsrcp, ddstp)

    return (pred[:E], gene_out_p[:N], cell_out_p[:N])


# speedup vs baseline: 1.1644x; 1.1644x over previous
import jax
import jax.numpy as jnp
from jax import lax
from jax.experimental import pallas as pl
from jax.experimental.pallas import tpu as pltpu
from jax.experimental.pallas import tpu_sc as plsc

N = 10000
NP = 10240
E = 320000
D = 128
R = 2
HALF = 64
NC = 2
NS = 16
L = 16
CH = 128
NCH = 160
ET = NCH * CH
EPAD = NS * ET
NCH32 = EPAD // (2 * NS) // CH
NSEG = 4
CHD = 64
NCHD = EPAD // (2 * NS) // CHD
SEGC = NCH // NSEG
SEGE = SEGC * CH


def _mesh():
    return plsc.VectorSubcoreMesh(
        core_axis_name="c", subcore_axis_name="s", num_cores=NC, num_subcores=NS
    )


def _deg_body(x3_hbm, y3_hbm, z_hbm, out_hbm,
              hist, Xb, Cb, ones, sa0, sb0, sa1, sb1):
    cid = lax.axis_index("c")
    sid = lax.axis_index("s")
    wid = cid * NS + sid
    for j in range(CH // L):
        ones[pl.ds(j * L, L)] = jnp.ones((L,), jnp.float32)

    zsl = pl.ds(sid * (4 * NP // NS), 4 * NP // NS)
    pltpu.sync_copy(z_hbm.at[zsl], hist.at[zsl])

    et32 = NCH32 * CH
    pltpu.sync_copy(x3_hbm.at[pl.ds(wid * et32, et32)], Xb)
    pltpu.sync_copy(y3_hbm.at[pl.ds(wid * et32, et32)], Cb)

    def cc(kk, carry):
        osl = pl.ds(kk * L, L)
        Cb[osl] = Cb[osl] + 2 * NP
        return carry

    lax.fori_loop(0, et32 // L, cc, 0)
    plsc.subcore_barrier()

    sa = (sa0, sa1)
    sb = (sb0, sb1)

    def pair(i2, carry):
        for b in range(2):
            c = i2 * 2 + b

            @pl.when(i2 >= 1)
            def _():
                pltpu.make_async_copy(
                    ones, hist.at[Xb.at[pl.ds((c - 2) * CH, CH)]], sa[b]).wait()
                pltpu.make_async_copy(
                    ones, hist.at[Cb.at[pl.ds((c - 2) * CH, CH)]], sb[b]).wait()

            pltpu.async_copy(ones, hist.at[Xb.at[pl.ds(c * CH, CH)]], sa[b],
                             add=True)
            pltpu.async_copy(ones, hist.at[Cb.at[pl.ds(c * CH, CH)]], sb[b],
                             add=True)
        return carry

    lax.fori_loop(0, NCH32 // 2, pair, 0)
    for b in range(2):
        c = NCH32 - 2 + b
        pltpu.make_async_copy(ones, hist.at[Xb.at[pl.ds(c * CH, CH)]],
                              sa[b]).wait()
        pltpu.make_async_copy(ones, hist.at[Cb.at[pl.ds(c * CH, CH)]],
                              sb[b]).wait()
    plsc.subcore_barrier()
    pltpu.sync_copy(hist.at[zsl],
                    out_hbm.at[pl.ds(cid * 4 * NP + sid * (4 * NP // NS),
                                     4 * NP // NS)])


def _deg_call(x, y):
    k = pl.kernel(
        _deg_body,
        out_type=jax.ShapeDtypeStruct((NC * 4 * NP,), jnp.float32),
        mesh=_mesh(),
        compiler_params=pltpu.CompilerParams(use_tc_tiling_on_sc=False),
        scratch_types=[
            pltpu.VMEM_SHARED((4 * NP,), jnp.float32),
            pltpu.VMEM((NCH32 * CH,), jnp.int32),
            pltpu.VMEM((NCH32 * CH,), jnp.int32),
            pltpu.VMEM((CH,), jnp.float32),
            pltpu.SemaphoreType.DMA,
            pltpu.SemaphoreType.DMA,
            pltpu.SemaphoreType.DMA,
            pltpu.SemaphoreType.DMA,
        ],
    )
    z = jnp.zeros((4 * NP,), jnp.float32)
    return k(x, y, z).reshape(NC, 4 * NP)


_TB = 256


def _table_body(feat_ref, w_ref, deg_ref, tab_ref, inv_ref):
    f = feat_ref[...]
    for r in range(R):
        d = deg_ref[r] + deg_ref[R + r]
        inv = lax.rsqrt(jnp.maximum(d, 1.0))
        inv_ref[r] = inv
        res = jnp.dot(f, w_ref[r], preferred_element_type=jnp.float32)
        res = res * inv[:, None]
        tab_ref[0, r] = res[:, :HALF]
        tab_ref[1, r] = res[:, HALF:]


def _table_call(feat_p, w, deg4):
    return pl.pallas_call(
        _table_body,
        grid=(NP // _TB,),
        in_specs=[
            pl.BlockSpec((_TB, D), lambda nb: (nb, 0)),
            pl.BlockSpec((R, D, D), lambda nb: (0, 0, 0)),
            pl.BlockSpec((2 * R, _TB), lambda nb: (0, nb)),
        ],
        out_specs=[
            pl.BlockSpec((2, R, _TB, HALF), lambda nb: (0, 0, nb, 0)),
            pl.BlockSpec((R, _TB), lambda nb: (0, nb)),
        ],
        out_shape=[
            jax.ShapeDtypeStruct((2, R, NP, HALF), jnp.float32),
            jax.ShapeDtypeStruct((R, NP), jnp.float32),
        ],
    )(feat_p, w, deg4)


def _agg_body(tc_hbm, tg_hbm, x_hbm, y_hbm, z_hbm, aggc_hbm, aggg_hbm,
              A, Zb, Sb2, rows0, rows1, rows2, rows3,
              gs0, gs1, gs2, gs3, ss0, ss1, ss2, ss3):
    cid = lax.axis_index("c")
    sid = lax.axis_index("s")
    tpw = 2 * NP // NS

    def zero_a():
        zsl = pl.ds(sid * tpw, tpw)
        pltpu.sync_copy(z_hbm.at[zsl], A.at[zsl])

    rowsb = (rows0, rows1, rows2, rows3)
    gsem = (gs0, gs1, gs2, gs3)
    ssem = (ss0, ss1, ss2, ss3)

    def run_phase(tab_hbm, g_hbm, s_hbm, out_hbm):
        def seg(s, carry):
            soff = sid * ET + s * SEGE
            pltpu.sync_copy(g_hbm.at[pl.ds(soff, SEGE)], Zb)
            pltpu.sync_copy(s_hbm.at[pl.ds(soff, SEGE)], Sb2)

            def cz(kk, c2):
                osl = pl.ds(kk * L, L)
                Zb[osl] = Zb[osl] + cid * (2 * NP)
                return c2

            lax.fori_loop(0, SEGE // L, cz, 0)

            for b in range(4):
                pltpu.async_copy(tab_hbm.at[Zb.at[pl.ds(b * CH, CH)]],
                                 rowsb[b], gsem[b])

            def quad(i4, c2):
                c0 = i4 * 4
                for b in range(4):
                    c = c0 + b
                    pltpu.make_async_copy(tab_hbm.at[Zb.at[pl.ds(c * CH, CH)]],
                                          rowsb[b], gsem[b]).wait()
                    pltpu.async_copy(rowsb[b], A.at[Sb2.at[pl.ds(c * CH, CH)]],
                                     ssem[b], add=True)

                @pl.when(i4 + 1 < SEGC // 4)
                def _():
                    for b in range(4):
                        c = c0 + b
                        pltpu.make_async_copy(
                            rowsb[b], A.at[Sb2.at[pl.ds(c * CH, CH)]],
                            ssem[b]).wait()
                        pltpu.async_copy(
                            tab_hbm.at[Zb.at[pl.ds((c + 4) * CH, CH)]],
                            rowsb[b], gsem[b])

                return c2

            lax.fori_loop(0, SEGC // 4, quad, 0)
            for b in range(4):
                c = SEGC - 4 + b
                pltpu.make_async_copy(rowsb[b],
                                      A.at[Sb2.at[pl.ds(c * CH, CH)]],
                                      ssem[b]).wait()
            return carry

        lax.fori_loop(0, NSEG, seg, 0)
        plsc.subcore_barrier()
        roff = sid * tpw
        pltpu.sync_copy(A.at[pl.ds(roff, tpw)],
                        out_hbm.at[pl.ds(cid * 2 * NP + roff, tpw)])
        plsc.subcore_barrier()

    zero_a()
    plsc.subcore_barrier()
    run_phase(tc_hbm, x_hbm, y_hbm, aggc_hbm)
    zero_a()
    plsc.subcore_barrier()
    run_phase(tg_hbm, y_hbm, x_hbm, aggg_hbm)


def _agg_call(tabc, tabg, x, y):
    out = jax.ShapeDtypeStruct((NC * 2 * NP, HALF), jnp.float32)
    k = pl.kernel(
        _agg_body,
        out_type=(out, out),
        mesh=_mesh(),
        compiler_params=pltpu.CompilerParams(use_tc_tiling_on_sc=False),
        scratch_types=[
            pltpu.VMEM_SHARED((2 * NP, HALF), jnp.float32),
            pltpu.VMEM((SEGE,), jnp.int32),
            pltpu.VMEM((SEGE,), jnp.int32),
            pltpu.VMEM((CH, HALF), jnp.float32),
            pltpu.VMEM((CH, HALF), jnp.float32),
            pltpu.VMEM((CH, HALF), jnp.float32),
            pltpu.VMEM((CH, HALF), jnp.float32),
            pltpu.SemaphoreType.DMA,
            pltpu.SemaphoreType.DMA,
            pltpu.SemaphoreType.DMA,
            pltpu.SemaphoreType.DMA,
            pltpu.SemaphoreType.DMA,
            pltpu.SemaphoreType.DMA,
            pltpu.SemaphoreType.DMA,
            pltpu.SemaphoreType.DMA,
        ],
    )
    z = jnp.zeros((2 * NP, HALF), jnp.float32)
    aggc, aggg = k(tabc, tabg, x, y, z)
    return (aggc.reshape(NC, R, NP, HALF), aggg.reshape(NC, R, NP, HALF))


def _proj_body(agg_ref, inv_ref, w_ref, b_ref, out_ref):
    parts = []
    for r in range(R):
        iv = inv_ref[r][:, None]
        for h in range(2):
            parts.append(jnp.maximum(agg_ref[h, r], 0.0) * iv)
    a = jnp.concatenate(parts, axis=1)
    acc = jnp.dot(a, w_ref[...], preferred_element_type=jnp.float32)
    out_ref[...] = jnp.maximum(acc + b_ref[...], 0.0)


def _proj_call(agg4, inv, wfc, bfc):
    return pl.pallas_call(
        _proj_body,
        grid=(NP // _TB,),
        in_specs=[
            pl.BlockSpec((2, R, _TB, HALF), lambda nb: (0, 0, nb, 0)),
            pl.BlockSpec((R, _TB), lambda nb: (0, nb)),
            pl.BlockSpec((R * D, D), lambda nb: (0, 0)),
            pl.BlockSpec((1, D), lambda nb: (0, 0)),
        ],
        out_specs=pl.BlockSpec((_TB, D), lambda nb: (nb, 0)),
        out_shape=jax.ShapeDtypeStruct((NP, D), jnp.float32),
    )(agg4, inv, wfc, bfc.reshape(1, D))


def _dec_body(g_hbm, c_hbm, s2_hbm, d2_hbm, out_hbm,
              Sb, Db, g0, g1, c0b, c1b, P, p0, p1, sg0, sg1, sc0, sc1, sw0, sw1):
    cid = lax.axis_index("c")
    sid = lax.axis_index("s")
    wid = cid * NS + sid
    base = wid * (NCHD * CHD)
    lanes = lax.broadcasted_iota(jnp.int32, (L,), 0)

    pltpu.sync_copy(s2_hbm.at[pl.ds(2 * base, 2 * NCHD * CHD)], Sb)
    pltpu.sync_copy(d2_hbm.at[pl.ds(2 * base, 2 * NCHD * CHD)], Db)

    gbuf = (g0, g1)
    cbuf = (c0b, c1b)
    pbuf = (p0, p1)
    sg = (sg0, sg1)
    sc = (sc0, sc1)
    sw = (sw0, sw1)
    ipc = 2 * CHD

    pltpu.async_copy(g_hbm.at[Sb.at[pl.ds(0, ipc)]], gbuf[0], sg[0])
    pltpu.async_copy(c_hbm.at[Db.at[pl.ds(0, ipc)]], cbuf[0], sc[0])
    pltpu.async_copy(g_hbm.at[Sb.at[pl.ds(ipc, ipc)]], gbuf[1], sg[1])
    pltpu.async_copy(c_hbm.at[Db.at[pl.ds(ipc, ipc)]], cbuf[1], sc[1])

    def pair(i2, carry):
        for b in range(2):
            c = i2 * 2 + b
            pltpu.make_async_copy(g_hbm.at[Sb.at[pl.ds(c * ipc, ipc)]],
                                  gbuf[b], sg[b]).wait()
            pltpu.make_async_copy(c_hbm.at[Db.at[pl.ds(c * ipc, ipc)]],
                                  cbuf[b], sc[b]).wait()

            def edot(e, carry2):
                acc = jnp.zeros((L,), jnp.float32)
                for h in range(2):
                    for j in range(HALF // L):
                        acc = acc + (gbuf[b][2 * e + h, pl.ds(j * L, L)]
                                     * cbuf[b][2 * e + h, pl.ds(j * L, L)])
                P[pl.ds(e * L, L)] = acc
                return carry2

            lax.fori_loop(0, CHD, edot, 0)

            @pl.when(i2 >= 1)
            def _():
                pltpu.make_async_copy(
                    pbuf[b], out_hbm.at[pl.ds(base + (c - 2) * CHD, CHD)],
                    sw[b]).wait()

            def red(g, carry2):
                acc = jnp.zeros((L,), jnp.float32)
                rowbase = (g * L + lanes) * L
                for j in range(L):
                    acc = acc + plsc.load_gather(P, [rowbase + j])
                pbuf[b][pl.ds(g * L, L)] = acc
                return carry2

            lax.fori_loop(0, CHD // L, red, 0)
            pltpu.async_copy(pbuf[b], out_hbm.at[pl.ds(base + c * CHD, CHD)],
                             sw[b])

            @pl.when(i2 + 1 < NCHD // 2)
            def _():
                pltpu.async_copy(g_hbm.at[Sb.at[pl.ds((c + 2) * ipc, ipc)]],
                                 gbuf[b], sg[b])
                pltpu.async_copy(c_hbm.at[Db.at[pl.ds((c + 2) * ipc, ipc)]],
                                 cbuf[b], sc[b])

        return carry

    lax.fori_loop(0, NCHD // 2, pair, 0)
    for b in range(2):
        c = NCHD - 2 + b
        pltpu.make_async_copy(pbuf[b], out_hbm.at[pl.ds(base + c * CHD, CHD)],
                              sw[b]).wait()


def _dec_call(gout, cout, ds2, dd2):
    k = pl.kernel(
        _dec_body,
        out_type=jax.ShapeDtypeStruct((EPAD,), jnp.float32),
        mesh=_mesh(),
        compiler_params=pltpu.CompilerParams(
            use_tc_tiling_on_sc=False, needs_layout_passes=False),
        scratch_types=[
            pltpu.VMEM((2 * NCHD * CHD,), jnp.int32),
            pltpu.VMEM((2 * NCHD * CHD,), jnp.int32),
            pltpu.VMEM((2 * CHD, HALF), jnp.float32),
            pltpu.VMEM((2 * CHD, HALF), jnp.float32),
            pltpu.VMEM((2 * CHD, HALF), jnp.float32),
            pltpu.VMEM((2 * CHD, HALF), jnp.float32),
            pltpu.VMEM((CHD * L,), jnp.float32),
            pltpu.VMEM((CHD,), jnp.float32),
            pltpu.VMEM((CHD,), jnp.float32),
            pltpu.SemaphoreType.DMA,
            pltpu.SemaphoreType.DMA,
            pltpu.SemaphoreType.DMA,
            pltpu.SemaphoreType.DMA,
            pltpu.SemaphoreType.DMA,
            pltpu.SemaphoreType.DMA,
        ],
    )
    g2 = gout.reshape(2 * NP, HALF)
    c2 = cout.reshape(2 * NP, HALF)
    return k(g2, c2, ds2, dd2)


def kernel(ufeat, ifeat, enc_edge_index, enc_edge_type, dec_edge_index,
           W_u, W_i, Wfc_u, bfc_u, Wfc_i, bfc_i):
    src = enc_edge_index[0].astype(jnp.int32)
    dst = enc_edge_index[1].astype(jnp.int32)
    typ = enc_edge_type.astype(jnp.int32)

    pad = jnp.full((EPAD - E,), N, jnp.int32)
    x = jnp.concatenate([typ * NP + src, pad])
    y = jnp.concatenate([typ * NP + dst, pad])

    degp = _deg_call(x, y)
    gdeg = degp[:, :2 * NP].reshape(2 * R, NP)
    cdeg = degp[:, 2 * NP:].reshape(2 * R, NP)

    ufp = jnp.pad(ufeat, ((0, NP - N), (0, 0)))
    ifp = jnp.pad(ifeat, ((0, NP - N), (0, 0)))

    tabc4, invg = _table_call(ufp, W_u, gdeg)
    tabg4, invc = _table_call(ifp, W_i, cdeg)

    aggc, aggg = _agg_call(tabc4.reshape(2 * R * NP, HALF),
                           tabg4.reshape(2 * R * NP, HALF), x, y)

    gene_out_p = _proj_call(aggg, invg, Wfc_u, bfc_u)
    cell_out_p = _proj_call(aggc, invc, Wfc_i, bfc_i)

    dsrcp = jnp.concatenate([dec_edge_index[0].astype(jnp.int32), pad])
    ddstp = jnp.concatenate([dec_edge_index[1].astype(jnp.int32), pad])
    ds2 = (2 * dsrcp[:, None] + jnp.arange(2, dtype=jnp.int32)).reshape(-1)
    dd2 = (2 * ddstp[:, None] + jnp.arange(2, dtype=jnp.int32)).reshape(-1)
    pred = _dec_call(gene_out_p, cell_out_p, ds2, dd2)

    return (pred[:E], gene_out_p[:N], cell_out_p[:N])
